# baseline (speedup 1.0000x reference)
import dataclasses
import functools

import jax
import jax.numpy as jnp
from jax import lax
from jax.experimental import pallas as pl
from jax.experimental.pallas import tpu as pltpu
from jax.experimental.pallas import tpu_sc as plsc

N = 10000
E = 320000
HID = 128

NC = 2
NS = 16
L = 16
NW = NC * NS

@functools.cache
def _mesh():
    return plsc.VectorSubcoreMesh(core_axis_name="c", subcore_axis_name="s",
                                  num_cores=NC, num_subcores=NS)

_CP = pltpu.CompilerParams()
if "needs_layout_passes" in pltpu.CompilerParams.__dataclass_fields__:
    _CP = dataclasses.replace(_CP, needs_layout_passes=False)


def _sc_gather_rows(table, idx, n_rows, chunk=64):
    D = table.shape[1]
    per_w = n_rows // NW
    n_chunks = per_w // chunk

    @functools.partial(
        pl.kernel,
        mesh=_mesh(),
        out_type=jax.ShapeDtypeStruct((n_rows, D), table.dtype),
        scratch_types=[
            pltpu.VMEM((chunk,), jnp.int32),
            pltpu.VMEM((chunk, D), table.dtype),
            pltpu.SemaphoreType.DMA,
        ],
    )
    def k(table_hbm, idx_hbm, out_hbm, idx_v, rows_v, sem):
        wid = lax.axis_index("s") * NC + lax.axis_index("c")
        base = wid * per_w

        @pl.loop(0, n_chunks)
        def _(j):
            off = base + j * chunk
            pltpu.sync_copy(idx_hbm.at[pl.ds(off, chunk)], idx_v)
            pltpu.async_copy(table_hbm.at[idx_v], rows_v, sem).wait()
            pltpu.sync_copy(rows_v, out_hbm.at[pl.ds(off, chunk)])

    return k(table, idx)


def _embed_rows(table, idx):
    n_pad = 10240
    idx_p = jnp.concatenate(
        [idx.astype(jnp.int32), jnp.zeros((n_pad - N,), jnp.int32)])
    rows = _sc_gather_rows(table, idx_p, n_pad, chunk=64)
    return rows[:N]


EP = 331776
E_CHUNK = 128
N_CHUNKS_W = EP // (NW * E_CHUNK)
PER_W = EP // NW
NPAD = 10240
NDEN = 10016


def _sc_gat_edges(h, hs, hd, src_flat, dst_chunks):
    @functools.partial(
        pl.kernel,
        mesh=_mesh(),
        out_type=[
            jax.ShapeDtypeStruct((2 * NPAD, HID), jnp.float32),
            jax.ShapeDtypeStruct((NW * NDEN,), jnp.float32),
        ],
        compiler_params=_CP,
        scratch_types=[
            pltpu.VMEM((N,), jnp.float32),
            pltpu.VMEM((N,), jnp.float32),
            pltpu.VMEM((NDEN,), jnp.float32),
            pltpu.VMEM((E_CHUNK,), jnp.int32),
            pltpu.VMEM((1, E_CHUNK), jnp.int32),
            pltpu.VMEM((E_CHUNK,), jnp.float32),
            pltpu.VMEM((E_CHUNK, HID), jnp.float32),
            pltpu.VMEM_SHARED((NPAD, HID), jnp.float32),
            pltpu.SemaphoreType.DMA,
        ],
    )
    def k(h_hbm, hs_hbm, hd_hbm, src_hbm, dchunk_hbm, num_hbm, den_hbm,
          hs_v, hd_v, den_v, src_v, dst_v, ex_v, rows_v, acc, sem):
        cidx = lax.axis_index("c")
        sid = lax.axis_index("s")
        wid = sid * NC + cidx
        base = wid * PER_W

        pltpu.sync_copy(hs_hbm, hs_v)
        pltpu.sync_copy(hd_hbm, hd_v)

        @pl.loop(0, NDEN // L)
        def _(t):
            den_v[pl.ds(t * L, L)] = jnp.zeros((L,), jnp.float32)

        @pl.loop(0, E_CHUNK)
        def _(r):
            for kk in range(HID // L):
                rows_v[r, pl.ds(kk * L, L)] = jnp.zeros((L,), jnp.float32)

        rows_per_tile = NPAD // NS
        @pl.loop(0, rows_per_tile // E_CHUNK)
        def _(t):
            pltpu.sync_copy(
                rows_v, acc.at[pl.ds(sid * rows_per_tile + t * E_CHUNK,
                                     E_CHUNK)])
        plsc.subcore_barrier()

        lane = lax.iota(jnp.int32, 16)
        nxt = jnp.minimum(lane + 1, 15)
        dnums = lax.GatherDimensionNumbers(
            offset_dims=(), collapsed_slice_dims=(0,), start_index_map=(0,))

        def shift_left(v):
            return lax.gather(v, nxt[:, None], dnums, slice_sizes=(1,),
                              mode=lax.GatherScatterMode.PROMISE_IN_BOUNDS)

        @pl.loop(0, N_CHUNKS_W)
        def _(j):
            off = base + j * E_CHUNK
            gcid = wid * N_CHUNKS_W + j
            pltpu.sync_copy(src_hbm.at[pl.ds(off, E_CHUNK)], src_v)
            cp = pltpu.async_copy(h_hbm.at[src_v], rows_v, sem)
            pltpu.sync_copy(dchunk_hbm.at[pl.ds(gcid, 1)], dst_v)
            for u in range(E_CHUNK // L):
                s16 = src_v[pl.ds(u * L, L)]
                d16 = dst_v[0, pl.ds(u * L, L)]
                dg = jnp.minimum(d16, N - 1)
                e = plsc.load_gather(hs_v, [s16]) + plsc.load_gather(hd_v, [dg])
                e = jnp.maximum(e, 0.2 * e)
                ex = jnp.exp(e)
                ex_v[pl.ds(u * L, L)] = ex
                kk, vv = plsc.sort_key_val(d16, ex)
                c = plsc.cumsum(vv)
                knx = shift_left(kk)
                bend = kk != knx
                plsc.addupdate_scatter(den_v, [kk], c,
                                       mask=bend | (lane == 15))
                plsc.addupdate_scatter(den_v, [knx], -c,
                                       mask=bend & (lane != 15))
            cp.wait()

            @plsc.parallel_loop(0, E_CHUNK // L, unroll=2)
            def _(g):
                exg = ex_v[pl.ds(g * L, L)]
                for i in range(L):
                    r = g * L + i
                    exr = exg[i]
                    for kk in range(HID // L):
                        rows_v[r, pl.ds(kk * L, L)] = (
                            rows_v[r, pl.ds(kk * L, L)] * exr)

            pltpu.sync_copy(rows_v, acc.at[dst_v.at[0]], add=True)

        pltpu.sync_copy(den_v, den_hbm.at[pl.ds(wid * NDEN, NDEN)])
        plsc.subcore_barrier()
        row0 = sid * rows_per_tile
        pltpu.sync_copy(acc.at[pl.ds(row0, rows_per_tile)],
                        num_hbm.at[pl.ds(cidx * NPAD + row0, rows_per_tile)])

    return k(h, hs, hd, src_flat, dst_chunks)


BR = 256


def _tc_stage_a(x, pe, wall, bias):
    def body(x_ref, pe_ref, w_ref, b_ref, o_ref):
        pe_b = pe_ref[...]
        nrm = jnp.sqrt(jnp.sum(pe_b * pe_b, axis=1, keepdims=True))
        pe_b = pe_b * jnp.minimum(1.0, 1.0 / jnp.maximum(nrm, 1e-7))
        acc = jnp.dot(x_ref[...], w_ref[:128, :],
                      preferred_element_type=jnp.float32)
        acc += jnp.dot(pe_b, w_ref[128:, :],
                       preferred_element_type=jnp.float32)
        o_ref[...] = acc + b_ref[...]

    return pl.pallas_call(
        body,
        grid=(NPAD // BR,),
        in_specs=[
            pl.BlockSpec((BR, 128), lambda i: (i, 0)),
            pl.BlockSpec((BR, 128), lambda i: (i, 0)),
            pl.BlockSpec((256, 384), lambda i: (0, 0)),
            pl.BlockSpec((1, 384), lambda i: (0, 0)),
        ],
        out_specs=pl.BlockSpec((BR, 384), lambda i: (i, 0)),
        out_shape=jax.ShapeDtypeStruct((NPAD, 384), jnp.float32),
    )(x, pe, wall, bias)


def _tc_stage_b(nacc, den_col, hall1, b1row, wall, bias):
    def body(n0_ref, n1_ref, d_ref, hall_ref, b1_ref, w_ref, bias_ref,
             h_ref, o_ref):
        num = n0_ref[...] + n1_ref[...]
        h = num / (d_ref[...] + 1e-16) + b1_ref[...] + hall_ref[:, 256:384]
        h = jnp.maximum(h, 0.0)
        h_ref[...] = h
        o_ref[...] = jnp.dot(h, w_ref[...],
                             preferred_element_type=jnp.float32) + bias_ref[...]

    nb = NPAD // BR
    return pl.pallas_call(
        body,
        grid=(nb,),
        in_specs=[
            pl.BlockSpec((BR, 128), lambda i: (i, 0)),
            pl.BlockSpec((BR, 128), lambda i: (i + nb, 0)),
            pl.BlockSpec((BR, 1), lambda i: (i, 0)),
            pl.BlockSpec((BR, 384), lambda i: (i, 0)),
            pl.BlockSpec((1, 128), lambda i: (0, 0)),
            pl.BlockSpec((128, 384), lambda i: (0, 0)),
            pl.BlockSpec((1, 384), lambda i: (0, 0)),
        ],
        out_specs=[
            pl.BlockSpec((BR, 128), lambda i: (i, 0)),
            pl.BlockSpec((BR, 384), lambda i: (i, 0)),
        ],
        out_shape=[
            jax.ShapeDtypeStruct((NPAD, 128), jnp.float32),
            jax.ShapeDtypeStruct((NPAD, 384), jnp.float32),
        ],
    )(nacc, nacc, den_col, hall1, b1row, wall, bias)


def _tc_stage_c(nacc, den_col, hall2, b2row, ctrl, pert_e,
                wm1a, wm1c, wm1b, bm1row, wm2p, bm2row):
    def body(n0_ref, n1_ref, d_ref, hall_ref, b2_ref, c_ref, p_ref,
             wa_ref, wc_ref, wb_ref, bm1_ref, w2_ref, bm2_ref, o_ref):
        num = n0_ref[...] + n1_ref[...]
        hf = num / (d_ref[...] + 1e-16) + b2_ref[...] + hall_ref[:, 256:384]
        a = jnp.dot(hf, wa_ref[...], preferred_element_type=jnp.float32)
        a += jnp.dot(p_ref[...], wb_ref[...],
                     preferred_element_type=jnp.float32)
        a += c_ref[...] * wc_ref[...]
        a = jnp.maximum(a + bm1_ref[...], 0.0)
        o = jnp.dot(a, w2_ref[...], preferred_element_type=jnp.float32)
        o_ref[...] = jnp.maximum(o + bm2_ref[...], 0.0)

    nb = NPAD // BR
    return pl.pallas_call(
        body,
        grid=(nb,),
        in_specs=[
            pl.BlockSpec((BR, 128), lambda i: (i, 0)),
            pl.BlockSpec((BR, 128), lambda i: (i + nb, 0)),
            pl.BlockSpec((BR, 1), lambda i: (i, 0)),
            pl.BlockSpec((BR, 384), lambda i: (i, 0)),
            pl.BlockSpec((1, 128), lambda i: (0, 0)),
            pl.BlockSpec((BR, 1), lambda i: (i, 0)),
            pl.BlockSpec((BR, 128), lambda i: (i, 0)),
            pl.BlockSpec((128, 64), lambda i: (0, 0)),
            pl.BlockSpec((1, 64), lambda i: (0, 0)),
            pl.BlockSpec((128, 64), lambda i: (0, 0)),
            pl.BlockSpec((1, 64), lambda i: (0, 0)),
            pl.BlockSpec((64, 128), lambda i: (0, 0)),
            pl.BlockSpec((1, 128), lambda i: (0, 0)),
        ],
        out_specs=pl.BlockSpec((BR, 128), lambda i: (i, 0)),
        out_shape=jax.ShapeDtypeStruct((NPAD, 128), jnp.float32),
    )(nacc, nacc, den_col, hall2, b2row, ctrl, pert_e,
      wm1a, wm1c, wm1b, bm1row, wm2p, bm2row)


def kernel(x, edge_index, edge_attr, pos, pert, ctrl, gene_table, pert_table,
           W1, a1s, a1d, b1, Wl1, bl1, W2, a2s, a2d, b2, Wl2, bl2,
           Wm1, bm1, Wm2, bm2):
    src, dst = edge_index[0], edge_index[1]

    pe_raw = _sc_gather_rows(
        gene_table,
        jnp.concatenate([pos.astype(jnp.int32),
                         jnp.zeros((NPAD - N,), jnp.int32)]),
        NPAD, chunk=64)
    pert_raw = _sc_gather_rows(
        pert_table,
        jnp.concatenate([pert.astype(jnp.int32),
                         jnp.zeros((NPAD - N,), jnp.int32)]),
        NPAD, chunk=64)

    loop = jnp.arange(N, dtype=jnp.int32)
    npad_e = EP - (E + N)
    s_p = jnp.concatenate([src.astype(jnp.int32), loop,
                           jnp.zeros((npad_e,), jnp.int32)])
    d_p = jnp.concatenate([dst.astype(jnp.int32), loop,
                           jnp.full((npad_e,), N, jnp.int32)])
    dst_chunks = d_p.reshape(EP // E_CHUNK, E_CHUNK)

    def augment(W, a_s, a_d, Wl, bl):
        wall = jnp.concatenate(
            [W, (W @ a_s)[:, None], (W @ a_d)[:, None],
             jnp.zeros((W.shape[0], 126), jnp.float32), Wl], axis=1)
        bias = jnp.concatenate(
            [jnp.zeros((256,), jnp.float32), bl])[None, :]
        return wall, bias

    w1all, bias1 = augment(W1, a1s, a1d, Wl1, bl1)
    w2all, bias2 = augment(W2, a2s, a2d, Wl2, bl2)

    x_p = jnp.pad(x, ((0, NPAD - N), (0, 0)))
    hall1 = _tc_stage_a(x_p, pe_raw, w1all, bias1)
    h1 = hall1[:, :128]
    hs1 = hall1[:N, 128]
    hd1 = hall1[:N, 129]

    nacc1, dacc1 = _sc_gat_edges(h1, hs1, hd1, s_p, dst_chunks)
    den1 = jnp.pad(dacc1.reshape(NW, NDEN).sum(axis=0),
                   (0, NPAD - NDEN))[:, None]

    h, hall2 = _tc_stage_b(nacc1, den1, hall1, b1[None, :], w2all, bias2)
    hs2 = hall2[:N, 128]
    hd2 = hall2[:N, 129]

    nacc2, dacc2 = _sc_gat_edges(hall2[:, :128], hs2, hd2, s_p, dst_chunks)
    den2 = jnp.pad(dacc2.reshape(NW, NDEN).sum(axis=0),
                   (0, NPAD - NDEN))[:, None]

    ctrl_p = jnp.pad(ctrl, ((0, NPAD - N), (0, 0)))
    wm2p = jnp.pad(Wm2, ((0, 0), (0, 127)))
    bm2row = jnp.broadcast_to(bm2, (1, 128)).astype(jnp.float32)
    out = _tc_stage_c(nacc2, den2, hall2, b2[None, :], ctrl_p, pert_raw,
                      Wm1[:128], Wm1[128:129], Wm1[129:], bm1[None, :],
                      wm2p, bm2row)
    return out[:N, 0]


# speedup vs baseline: 1.2748x; 1.2748x over previous
import dataclasses
import functools

import jax
import jax.numpy as jnp
from jax import lax
from jax.experimental import pallas as pl
from jax.experimental.pallas import tpu as pltpu
from jax.experimental.pallas import tpu_sc as plsc

N = 10000
E = 320000
HID = 128

NC = 2
NS = 16
L = 16
NW = NC * NS

@functools.cache
def _mesh():
    return plsc.VectorSubcoreMesh(core_axis_name="c", subcore_axis_name="s",
                                  num_cores=NC, num_subcores=NS)

_CP = pltpu.CompilerParams()
if "needs_layout_passes" in pltpu.CompilerParams.__dataclass_fields__:
    _CP = dataclasses.replace(_CP, needs_layout_passes=False)


def _sc_gather_rows(table, idx, n_rows, chunk=64):
    D = table.shape[1]
    per_w = n_rows // NW
    n_chunks = per_w // chunk

    @functools.partial(
        pl.kernel,
        mesh=_mesh(),
        out_type=jax.ShapeDtypeStruct((n_rows, D), table.dtype),
        scratch_types=[
            pltpu.VMEM((chunk,), jnp.int32),
            pltpu.VMEM((chunk, D), table.dtype),
            pltpu.SemaphoreType.DMA,
        ],
    )
    def k(table_hbm, idx_hbm, out_hbm, idx_v, rows_v, sem):
        wid = lax.axis_index("s") * NC + lax.axis_index("c")
        base = wid * per_w

        @pl.loop(0, n_chunks)
        def _(j):
            off = base + j * chunk
            pltpu.sync_copy(idx_hbm.at[pl.ds(off, chunk)], idx_v)
            pltpu.async_copy(table_hbm.at[idx_v], rows_v, sem).wait()
            pltpu.sync_copy(rows_v, out_hbm.at[pl.ds(off, chunk)])

    return k(table, idx)


def _embed_rows(table, idx):
    n_pad = 10240
    idx_p = jnp.concatenate(
        [idx.astype(jnp.int32), jnp.zeros((n_pad - N,), jnp.int32)])
    rows = _sc_gather_rows(table, idx_p, n_pad, chunk=64)
    return rows[:N]


EP = 331776
E_CHUNK = 96
N_CHUNKS_W = EP // (NW * E_CHUNK)
PER_W = EP // NW
NPAD = 10240
NDEN = 10016


def _sc_gat_logits(hs, hd, src_flat, dst_flat):
    @functools.partial(
        pl.kernel,
        mesh=_mesh(),
        out_type=[
            jax.ShapeDtypeStruct((EP,), jnp.float32),
            jax.ShapeDtypeStruct((NW * NDEN,), jnp.float32),
        ],
        compiler_params=_CP,
        scratch_types=[
            pltpu.VMEM((N,), jnp.float32),
            pltpu.VMEM((N,), jnp.float32),
            pltpu.VMEM((NDEN,), jnp.float32),
            pltpu.VMEM((PER_W,), jnp.int32),
            pltpu.VMEM((PER_W,), jnp.int32),
            pltpu.VMEM((PER_W,), jnp.float32),
            pltpu.SemaphoreType.DMA,
            pltpu.SemaphoreType.DMA,
            pltpu.SemaphoreType.DMA,
            pltpu.SemaphoreType.DMA,
        ],
    )
    def k(hs_hbm, hd_hbm, src_hbm, dst_hbm, ex_hbm, den_hbm,
          hs_v, hd_v, den_v, src_all, dst_all, ex_all,
          sem0, sem1, sem2, sem3):
        cidx = lax.axis_index("c")
        sid = lax.axis_index("s")
        wid = sid * NC + cidx
        base = wid * PER_W

        c0 = pltpu.async_copy(hs_hbm, hs_v, sem0)
        c1 = pltpu.async_copy(hd_hbm, hd_v, sem1)
        c2 = pltpu.async_copy(src_hbm.at[pl.ds(base, PER_W)], src_all, sem2)
        c3 = pltpu.async_copy(dst_hbm.at[pl.ds(base, PER_W)], dst_all, sem3)

        @pl.loop(0, NDEN // L)
        def _(t):
            den_v[pl.ds(t * L, L)] = jnp.zeros((L,), jnp.float32)

        c0.wait()
        c1.wait()
        c2.wait()
        c3.wait()

        lane = lax.iota(jnp.int32, 16)
        nxt = jnp.minimum(lane + 1, 15)
        dnums = lax.GatherDimensionNumbers(
            offset_dims=(), collapsed_slice_dims=(0,), start_index_map=(0,))

        def shift_left(v):
            return lax.gather(v, nxt[:, None], dnums, slice_sizes=(1,),
                              mode=lax.GatherScatterMode.PROMISE_IN_BOUNDS)

        @pl.loop(0, PER_W // L, unroll=4)
        def _(g):
            s16 = src_all[pl.ds(g * L, L)]
            d16 = dst_all[pl.ds(g * L, L)]
            dg = jnp.minimum(d16, N - 1)
            e = plsc.load_gather(hs_v, [s16]) + plsc.load_gather(hd_v, [dg])
            e = jnp.maximum(e, 0.2 * e)
            ex = jnp.exp(e)
            ex_all[pl.ds(g * L, L)] = ex
            kk, vv = plsc.sort_key_val(d16, ex)
            c = plsc.cumsum(vv)
            knx = shift_left(kk)
            bend = kk != knx
            plsc.addupdate_scatter(den_v, [kk], c, mask=bend | (lane == 15))
            plsc.addupdate_scatter(den_v, [knx], -c, mask=bend & (lane != 15))

        pltpu.sync_copy(ex_all, ex_hbm.at[pl.ds(base, PER_W)])
        pltpu.sync_copy(den_v, den_hbm.at[pl.ds(wid * NDEN, NDEN)])

    return k(hs, hd, src_flat, dst_flat)


def _sc_gat_messages(h, ex_flat, src_flat, dst_chunks):
    CK = E_CHUNK
    NCH = N_CHUNKS_W

    @functools.partial(
        pl.kernel,
        mesh=_mesh(),
        out_type=jax.ShapeDtypeStruct((2 * NPAD, HID), jnp.float32),
        compiler_params=_CP,
        scratch_types=(
            [pltpu.VMEM((CK,), jnp.int32)] * 3 +
            [pltpu.VMEM((CK,), jnp.float32)] * 3 +
            [pltpu.VMEM((1, CK), jnp.int32)] * 3 +
            [pltpu.VMEM((CK, HID), jnp.float32)] * 2 +
            [pltpu.VMEM_SHARED((NPAD, HID), jnp.float32)] +
            [pltpu.SemaphoreType.DMA] * 7
        ),
    )
    def k(h_hbm, ex_hbm, src_hbm, dchunk_hbm, num_hbm,
          s0, s1, s2, e0, e1, e2, d0, d1, d2, r0, r1, acc,
          si0, si1, si2, sg0, sg1, ss0, ss1):
        srcs, exs, dsts = [s0, s1, s2], [e0, e1, e2], [d0, d1, d2]
        rows, semi = [r0, r1], [si0, si1, si2]
        semg, sems = [sg0, sg1], [ss0, ss1]
        cidx = lax.axis_index("c")
        sid = lax.axis_index("s")
        wid = sid * NC + cidx
        base = wid * PER_W

        def idx_descs(jj, bi):
            off = base + jj * CK
            gcid = wid * NCH + jj
            return [
                pltpu.make_async_copy(src_hbm.at[pl.ds(off, CK)], srcs[bi],
                                      semi[bi]),
                pltpu.make_async_copy(ex_hbm.at[pl.ds(off, CK)], exs[bi],
                                      semi[bi]),
                pltpu.make_async_copy(dchunk_hbm.at[pl.ds(gcid, 1)], dsts[bi],
                                      semi[bi]),
            ]

        def gather_desc(b, bi):
            return pltpu.make_async_copy(h_hbm.at[srcs[bi]], rows[b], semg[b])

        def scatter_desc(b, bi):
            return pltpu.make_async_copy(rows[b], acc.at[dsts[bi].at[0]],
                                         sems[b])

        @pl.loop(0, CK)
        def _(r):
            for kk in range(HID // L):
                r0[r, pl.ds(kk * L, L)] = jnp.zeros((L,), jnp.float32)

        rows_per_tile = NPAD // NS
        row_base = sid * rows_per_tile
        for t in range(6):
            pltpu.sync_copy(r0, acc.at[pl.ds(row_base + t * CK, CK)])
        pltpu.sync_copy(r0.at[pl.ds(0, 64)],
                        acc.at[pl.ds(row_base + 6 * CK, 64)])
        plsc.subcore_barrier()

        for dsc in idx_descs(0, 0):
            dsc.start()
        for dsc in idx_descs(1, 1):
            dsc.start()
        for dsc in idx_descs(0, 0):
            dsc.wait()
        gather_desc(0, 0).start()

        @pl.loop(0, NCH // 6)
        def _(t):
            for c in range(6):
                j = t * 6 + c
                b, bn, bi = c % 2, (c + 1) % 2, c % 3
                bi1, bi2 = (c + 1) % 3, (c + 2) % 3

                @pl.when(j >= 1)
                def _():
                    scatter_desc(bn, (c + 5) % 3).wait()

                @pl.when(j + 2 < NCH)
                def _():
                    for dsc in idx_descs(j + 2, bi2):
                        dsc.start()

                @pl.when(j + 1 < NCH)
                def _():
                    for dsc in idx_descs(j + 1, bi1):
                        dsc.wait()
                    gather_desc(bn, bi1).start()

                gather_desc(b, bi).wait()

                exv = exs[bi]
                rv = rows[b]

                @plsc.parallel_loop(0, CK // L, unroll=2)
                def _(g):
                    exg = exv[pl.ds(g * L, L)]
                    for i in range(L):
                        r = g * L + i
                        exr = exg[i]
                        for kk in range(HID // L):
                            rv[r, pl.ds(kk * L, L)] = (
                                rv[r, pl.ds(kk * L, L)] * exr)

                pltpu.async_copy(rv, acc.at[dsts[bi].at[0]], sems[b],
                                 add=True)

        scatter_desc((NCH - 1) % 2, (NCH - 1) % 3).wait()
        plsc.subcore_barrier()
        pltpu.sync_copy(acc.at[pl.ds(row_base, rows_per_tile)],
                        num_hbm.at[pl.ds(cidx * NPAD + row_base,
                                         rows_per_tile)])

    return k(h, ex_flat, src_flat, dst_chunks)


def _sc_gat_edges(h, hs, hd, src_flat, dst_flat, dst_chunks):
    ex_flat, dacc = _sc_gat_logits(hs, hd, src_flat, dst_flat)
    nacc = _sc_gat_messages(h, ex_flat, src_flat, dst_chunks)
    return nacc, dacc


BR = 256


def _tc_stage_a(x, pe, wall, bias):
    def body(x_ref, pe_ref, w_ref, b_ref, o_ref):
        pe_b = pe_ref[...]
        nrm = jnp.sqrt(jnp.sum(pe_b * pe_b, axis=1, keepdims=True))
        pe_b = pe_b * jnp.minimum(1.0, 1.0 / jnp.maximum(nrm, 1e-7))
        acc = jnp.dot(x_ref[...], w_ref[:128, :],
                      preferred_element_type=jnp.float32)
        acc += jnp.dot(pe_b, w_ref[128:, :],
                       preferred_element_type=jnp.float32)
        o_ref[...] = acc + b_ref[...]

    return pl.pallas_call(
        body,
        grid=(NPAD // BR,),
        in_specs=[
            pl.BlockSpec((BR, 128), lambda i: (i, 0)),
            pl.BlockSpec((BR, 128), lambda i: (i, 0)),
            pl.BlockSpec((256, 384), lambda i: (0, 0)),
            pl.BlockSpec((1, 384), lambda i: (0, 0)),
        ],
        out_specs=pl.BlockSpec((BR, 384), lambda i: (i, 0)),
        out_shape=jax.ShapeDtypeStruct((NPAD, 384), jnp.float32),
    )(x, pe, wall, bias)


def _tc_stage_b(nacc, den_col, hall1, b1row, wall, bias):
    def body(n0_ref, n1_ref, d_ref, hall_ref, b1_ref, w_ref, bias_ref,
             h_ref, o_ref):
        num = n0_ref[...] + n1_ref[...]
        h = num / (d_ref[...] + 1e-16) + b1_ref[...] + hall_ref[:, 256:384]
        h = jnp.maximum(h, 0.0)
        h_ref[...] = h
        o_ref[...] = jnp.dot(h, w_ref[...],
                             preferred_element_type=jnp.float32) + bias_ref[...]

    nb = NPAD // BR
    return pl.pallas_call(
        body,
        grid=(nb,),
        in_specs=[
            pl.BlockSpec((BR, 128), lambda i: (i, 0)),
            pl.BlockSpec((BR, 128), lambda i: (i + nb, 0)),
            pl.BlockSpec((BR, 1), lambda i: (i, 0)),
            pl.BlockSpec((BR, 384), lambda i: (i, 0)),
            pl.BlockSpec((1, 128), lambda i: (0, 0)),
            pl.BlockSpec((128, 384), lambda i: (0, 0)),
            pl.BlockSpec((1, 384), lambda i: (0, 0)),
        ],
        out_specs=[
            pl.BlockSpec((BR, 128), lambda i: (i, 0)),
            pl.BlockSpec((BR, 384), lambda i: (i, 0)),
        ],
        out_shape=[
            jax.ShapeDtypeStruct((NPAD, 128), jnp.float32),
            jax.ShapeDtypeStruct((NPAD, 384), jnp.float32),
        ],
    )(nacc, nacc, den_col, hall1, b1row, wall, bias)


def _tc_stage_c(nacc, den_col, hall2, b2row, ctrl, pert_e,
                wm1a, wm1c, wm1b, bm1row, wm2p, bm2row):
    def body(n0_ref, n1_ref, d_ref, hall_ref, b2_ref, c_ref, p_ref,
             wa_ref, wc_ref, wb_ref, bm1_ref, w2_ref, bm2_ref, o_ref):
        num = n0_ref[...] + n1_ref[...]
        hf = num / (d_ref[...] + 1e-16) + b2_ref[...] + hall_ref[:, 256:384]
        a = jnp.dot(hf, wa_ref[...], preferred_element_type=jnp.float32)
        a += jnp.dot(p_ref[...], wb_ref[...],
                     preferred_element_type=jnp.float32)
        a += c_ref[...] * wc_ref[...]
        a = jnp.maximum(a + bm1_ref[...], 0.0)
        o = jnp.dot(a, w2_ref[...], preferred_element_type=jnp.float32)
        o_ref[...] = jnp.maximum(o + bm2_ref[...], 0.0)

    nb = NPAD // BR
    return pl.pallas_call(
        body,
        grid=(nb,),
        in_specs=[
            pl.BlockSpec((BR, 128), lambda i: (i, 0)),
            pl.BlockSpec((BR, 128), lambda i: (i + nb, 0)),
            pl.BlockSpec((BR, 1), lambda i: (i, 0)),
            pl.BlockSpec((BR, 384), lambda i: (i, 0)),
            pl.BlockSpec((1, 128), lambda i: (0, 0)),
            pl.BlockSpec((BR, 1), lambda i: (i, 0)),
            pl.BlockSpec((BR, 128), lambda i: (i, 0)),
            pl.BlockSpec((128, 64), lambda i: (0, 0)),
            pl.BlockSpec((1, 64), lambda i: (0, 0)),
            pl.BlockSpec((128, 64), lambda i: (0, 0)),
            pl.BlockSpec((1, 64), lambda i: (0, 0)),
            pl.BlockSpec((64, 128), lambda i: (0, 0)),
            pl.BlockSpec((1, 128), lambda i: (0, 0)),
        ],
        out_specs=pl.BlockSpec((BR, 128), lambda i: (i, 0)),
        out_shape=jax.ShapeDtypeStruct((NPAD, 128), jnp.float32),
    )(nacc, nacc, den_col, hall2, b2row, ctrl, pert_e,
      wm1a, wm1c, wm1b, bm1row, wm2p, bm2row)


def kernel(x, edge_index, edge_attr, pos, pert, ctrl, gene_table, pert_table,
           W1, a1s, a1d, b1, Wl1, bl1, W2, a2s, a2d, b2, Wl2, bl2,
           Wm1, bm1, Wm2, bm2):
    src, dst = edge_index[0], edge_index[1]

    pe_raw = _sc_gather_rows(
        gene_table,
        jnp.concatenate([pos.astype(jnp.int32),
                         jnp.zeros((NPAD - N,), jnp.int32)]),
        NPAD, chunk=64)
    pert_raw = _sc_gather_rows(
        pert_table,
        jnp.concatenate([pert.astype(jnp.int32),
                         jnp.zeros((NPAD - N,), jnp.int32)]),
        NPAD, chunk=64)

    loop = jnp.arange(N, dtype=jnp.int32)
    npad_e = EP - (E + N)
    s_p = jnp.concatenate([src.astype(jnp.int32), loop,
                           jnp.zeros((npad_e,), jnp.int32)])
    d_p = jnp.concatenate([dst.astype(jnp.int32), loop,
                           jnp.full((npad_e,), N, jnp.int32)])
    dst_chunks = d_p.reshape(EP // E_CHUNK, E_CHUNK)

    def augment(W, a_s, a_d, Wl, bl):
        wall = jnp.concatenate(
            [W, (W @ a_s)[:, None], (W @ a_d)[:, None],
             jnp.zeros((W.shape[0], 126), jnp.float32), Wl], axis=1)
        bias = jnp.concatenate(
            [jnp.zeros((256,), jnp.float32), bl])[None, :]
        return wall, bias

    w1all, bias1 = augment(W1, a1s, a1d, Wl1, bl1)
    w2all, bias2 = augment(W2, a2s, a2d, Wl2, bl2)

    x_p = jnp.pad(x, ((0, NPAD - N), (0, 0)))
    hall1 = _tc_stage_a(x_p, pe_raw, w1all, bias1)
    h1 = hall1[:, :128]
    hs1 = hall1[:N, 128]
    hd1 = hall1[:N, 129]

    nacc1, dacc1 = _sc_gat_edges(h1, hs1, hd1, s_p, d_p, dst_chunks)
    den1 = jnp.pad(dacc1.reshape(NW, NDEN).sum(axis=0),
                   (0, NPAD - NDEN))[:, None]

    h, hall2 = _tc_stage_b(nacc1, den1, hall1, b1[None, :], w2all, bias2)
    hs2 = hall2[:N, 128]
    hd2 = hall2[:N, 129]

    nacc2, dacc2 = _sc_gat_edges(hall2[:, :128], hs2, hd2, s_p, d_p,
                                 dst_chunks)
    den2 = jnp.pad(dacc2.reshape(NW, NDEN).sum(axis=0),
                   (0, NPAD - NDEN))[:, None]

    ctrl_p = jnp.pad(ctrl, ((0, NPAD - N), (0, 0)))
    wm2p = jnp.pad(Wm2, ((0, 0), (0, 127)))
    bm2row = jnp.broadcast_to(bm2, (1, 128)).astype(jnp.float32)
    out = _tc_stage_c(nacc2, den2, hall2, b2[None, :], ctrl_p, pert_raw,
                      Wm1[:128], Wm1[128:129], Wm1[129:], bm1[None, :],
                      wm2p, bm2row)
    return out[:N, 0]
